# baseline (speedup 1.0000x reference)
import jax
import jax.numpy as jnp
from jax import lax
from jax.experimental import pallas as pl
from jax.experimental.pallas import tpu as pltpu
from jax.experimental.pallas import tpu_sc as plsc

B, C, E, P = 16, 8, 256, 1024
ROWS = B * C * E
NC, NS = 2, 16
NW = NC * NS
B_SC = 12
BT = B - B_SC
PANELS_PER_W = (B_SC * C) // NW
R = 16
NCHUNK = PANELS_PER_W * E // R
NBUF = 3
LANES = 16
UNROLL = 4
assert NCHUNK % NBUF == 0


def _body(x_hbm, idx_hbm, o_hbm, idx_v, in0, in1, in2, out0, out1, out2,
          si0, si1, si2, so0, so1, so2):
    w = lax.axis_index("s") * NC + lax.axis_index("c")
    ins, outs = (in0, in1, in2), (out0, out1, out2)
    sin, sout = (si0, si1, si2), (so0, so1, so2)

    for k in range(PANELS_PER_W):
        ch = lax.rem(w * PANELS_PER_W + k, C)
        pltpu.sync_copy(idx_hbm.at[pl.ds(ch * P, P)],
                        idx_v.at[pl.ds(k * P, P)])

    wbase = w * NCHUNK

    def start_in(g, b):
        pltpu.async_copy(x_hbm.at[pl.ds((wbase + g) * R, R)], ins[b], sin[b])

    def start_out(g, b):
        pltpu.async_copy(outs[b], o_hbm.at[pl.ds((wbase + g) * R, R)],
                         sout[b])

    def wait_in(b):
        pltpu.make_async_copy(x_hbm.at[pl.ds(0, R)], ins[b], sin[b]).wait()

    def wait_out(b):
        pltpu.make_async_copy(outs[b], o_hbm.at[pl.ds(0, R)], sout[b]).wait()

    def gather(g, b):
        ibase = lax.div(g, E // R) * P

        @plsc.parallel_loop(0, P // LANES, unroll=UNROLL)
        def _(pc):
            off = pc * LANES
            iv = idx_v[pl.ds(ibase + off, LANES)]
            for r in range(R):
                rv = jnp.full((LANES,), r, jnp.int32)
                outs[b][r, pl.ds(off, LANES)] = plsc.load_gather(
                    ins[b], [rv, iv])

    for b in range(NBUF):
        start_in(b, b)

    def loop_body(j, _):
        for b in range(NBUF):
            g = j * NBUF + b
            wait_in(b)
            pl.when(j >= 1)(lambda b=b: wait_out(b))
            gather(g, b)
            start_out(g, b)
            pl.when(j < NCHUNK // NBUF - 1)(
                lambda g=g, b=b: start_in(g + NBUF, b))
        return 0

    lax.fori_loop(0, NCHUNK // NBUF, loop_body, 0)
    for b in range(NBUF):
        wait_out(b)


def _tc_body(x_ref, m_ref, o_ref):
    o_ref[0, 0] = jnp.dot(x_ref[0, 0].astype(jnp.bfloat16), m_ref[0],
                          preferred_element_type=jnp.float32)


@jax.jit
def _run(X, x2d, idx_flat, m_all):
    mesh = plsc.VectorSubcoreMesh(core_axis_name="c", subcore_axis_name="s")
    out_sc = pl.kernel(
        _body,
        out_type=jax.ShapeDtypeStruct((ROWS, P), jnp.float32),
        mesh=mesh,
        compiler_params=pltpu.CompilerParams(needs_layout_passes=False,
                                             use_tc_tiling_on_sc=True),
        scratch_types=[
            pltpu.VMEM((PANELS_PER_W * P,), jnp.int32),
            pltpu.VMEM((R, P), jnp.float32),
            pltpu.VMEM((R, P), jnp.float32),
            pltpu.VMEM((R, P), jnp.float32),
            pltpu.VMEM((R, P), jnp.float32),
            pltpu.VMEM((R, P), jnp.float32),
            pltpu.VMEM((R, P), jnp.float32),
            pltpu.SemaphoreType.DMA,
            pltpu.SemaphoreType.DMA,
            pltpu.SemaphoreType.DMA,
            pltpu.SemaphoreType.DMA,
            pltpu.SemaphoreType.DMA,
            pltpu.SemaphoreType.DMA,
        ],
    )(x2d, idx_flat)

    out_tc = pl.pallas_call(
        _tc_body,
        grid=(C, BT),
        in_specs=[
            pl.BlockSpec((1, 1, E, P), lambda c, b: (B_SC + b, c, 0, 0)),
            pl.BlockSpec((1, P, P), lambda c, b: (c, 0, 0)),
        ],
        out_specs=pl.BlockSpec((1, 1, E, P), lambda c, b: (b, c, 0, 0)),
        out_shape=jax.ShapeDtypeStruct((BT, C, E, P), jnp.float32),
    )(X, m_all)

    out4 = out_sc.reshape(B, C, E, P)
    return lax.dynamic_update_slice(out4, out_tc, (B_SC, 0, 0, 0))


def kernel(X, shuffled_idx, rand_idx):
    idx = lax.dynamic_index_in_dim(shuffled_idx, rand_idx, 0, keepdims=False)
    m_all = (jnp.arange(P, dtype=jnp.int32)[None, :, None]
             == idx[:, None, :]).astype(jnp.bfloat16)
    return _run(X, X.reshape(ROWS, P), idx.reshape(C * P), m_all)


# speedup vs baseline: 1.2194x; 1.2194x over previous
import jax
import jax.numpy as jnp
from jax import lax
from jax.experimental import pallas as pl
from jax.experimental.pallas import tpu as pltpu
from jax.experimental.pallas import tpu_sc as plsc

B, C, E, P = 16, 8, 256, 1024
ROWS = B * C * E
NC, NS = 2, 16
NW = NC * NS
PANELS_PER_W = (B * C) // NW
R = 16
NCHUNK = PANELS_PER_W * E // R
NBUF = 3
LANES = 16
UNROLL = 4


def _body(x_hbm, sidx_hbm, rand_hbm, o_hbm, rand_sm, idx_v,
          in0, in1, in2, out0, out1, out2,
          si0, si1, si2, so0, so1, so2):
    w = lax.axis_index("s") * NC + lax.axis_index("c")
    ins, outs = (in0, in1, in2), (out0, out1, out2)
    sin, sout = (si0, si1, si2), (so0, so1, so2)

    pltpu.sync_copy(rand_hbm, rand_sm)
    rand = rand_sm[...][0]
    pltpu.sync_copy(sidx_hbm.at[rand], idx_v)

    wbase = w * NCHUNK

    def start_in(g, b):
        pltpu.async_copy(x_hbm.at[pl.ds((wbase + g) * R, R)], ins[b], sin[b])

    def start_out(g, b):
        pltpu.async_copy(outs[b], o_hbm.at[pl.ds((wbase + g) * R, R)],
                         sout[b])

    def wait_in(b):
        pltpu.make_async_copy(x_hbm.at[pl.ds(0, R)], ins[b], sin[b]).wait()

    def wait_out(b):
        pltpu.make_async_copy(outs[b], o_hbm.at[pl.ds(0, R)], sout[b]).wait()

    def gather(g, b):
        ch = lax.rem(w * PANELS_PER_W + lax.div(g, E // R), C)

        @plsc.parallel_loop(0, P // LANES, unroll=UNROLL)
        def _(pc):
            off = pc * LANES
            iv = idx_v[ch, pl.ds(off, LANES)]
            for r in range(R):
                rv = jnp.full((LANES,), r, jnp.int32)
                outs[b][r, pl.ds(off, LANES)] = plsc.load_gather(
                    ins[b], [rv, iv])

    for b in range(NBUF):
        start_in(b, b)
    wait_in(0)
    gather(0, 0)
    start_out(0, 0)
    start_in(NBUF, 0)

    def loop_body(j, _):
        for i in range(NBUF):
            g = 1 + j * NBUF + i
            b = (1 + i) % NBUF
            wait_in(b)
            if i == NBUF - 1:
                wait_out(b)
            else:
                pl.when(j >= 1)(lambda b=b: wait_out(b))
            gather(g, b)
            start_out(g, b)
            pl.when(j < (NCHUNK - 1) // NBUF - 1)(
                lambda g=g, b=b: start_in(g + NBUF, b))
        return 0

    lax.fori_loop(0, (NCHUNK - 1) // NBUF, loop_body, 0)
    for b in (1, 2, 0):
        wait_out(b)


@jax.jit
def _run(x2d, sidx, rand1):
    mesh = plsc.VectorSubcoreMesh(core_axis_name="c", subcore_axis_name="s")
    return pl.kernel(
        _body,
        out_type=jax.ShapeDtypeStruct((ROWS, P), jnp.float32),
        mesh=mesh,
        compiler_params=pltpu.CompilerParams(needs_layout_passes=False,
                                             use_tc_tiling_on_sc=True),
        scratch_types=[
            pltpu.VMEM((LANES,), jnp.int32),
            pltpu.VMEM((C, P), jnp.int32),
            pltpu.VMEM((R, P), jnp.float32),
            pltpu.VMEM((R, P), jnp.float32),
            pltpu.VMEM((R, P), jnp.float32),
            pltpu.VMEM((R, P), jnp.float32),
            pltpu.VMEM((R, P), jnp.float32),
            pltpu.VMEM((R, P), jnp.float32),
            pltpu.SemaphoreType.DMA,
            pltpu.SemaphoreType.DMA,
            pltpu.SemaphoreType.DMA,
            pltpu.SemaphoreType.DMA,
            pltpu.SemaphoreType.DMA,
            pltpu.SemaphoreType.DMA,
        ],
    )(x2d, sidx, rand1)


def kernel(X, shuffled_idx, rand_idx):
    rand1 = jnp.full((LANES,), rand_idx, jnp.int32)
    out = _run(X.reshape(ROWS, P), shuffled_idx, rand1)
    return out.reshape(B, C, E, P)


# speedup vs baseline: 1.2366x; 1.0141x over previous
import jax
import jax.numpy as jnp
from jax import lax
from jax.experimental import pallas as pl
from jax.experimental.pallas import tpu as pltpu
from jax.experimental.pallas import tpu_sc as plsc

B, C, E, P = 16, 8, 256, 1024
ROWS = B * C * E
NC, NS = 2, 16
NW = NC * NS
PANELS_PER_W = (B * C) // NW
R = 8
NCHUNK = PANELS_PER_W * E // R
NBUF = 4
LANES = 16
UNROLL = 4
assert NCHUNK % NBUF == 0


def _body(x_hbm, sidx_hbm, rand_hbm, o_hbm, rand_sm, idx_v,
          in0, in1, in2, in3, out0, out1, out2, out3,
          si0, si1, si2, si3, so0, so1, so2, so3):
    w = lax.axis_index("s") * NC + lax.axis_index("c")
    ins, outs = (in0, in1, in2, in3), (out0, out1, out2, out3)
    sin, sout = (si0, si1, si2, si3), (so0, so1, so2, so3)

    pltpu.sync_copy(rand_hbm, rand_sm)
    rand = rand_sm[...][0]
    pltpu.sync_copy(sidx_hbm.at[rand], idx_v)

    wbase = w * NCHUNK

    def start_in(g, b):
        pltpu.async_copy(x_hbm.at[pl.ds((wbase + g) * R, R)], ins[b], sin[b])

    def start_out(g, b):
        pltpu.async_copy(outs[b], o_hbm.at[pl.ds((wbase + g) * R, R)],
                         sout[b])

    def wait_in(b):
        pltpu.make_async_copy(x_hbm.at[pl.ds(0, R)], ins[b], sin[b]).wait()

    def wait_out(b):
        pltpu.make_async_copy(outs[b], o_hbm.at[pl.ds(0, R)], sout[b]).wait()

    def gather(g, b):
        ch = lax.rem(w * PANELS_PER_W + lax.div(g, E // R), C)

        @plsc.parallel_loop(0, P // LANES, unroll=UNROLL)
        def _(pc):
            off = pc * LANES
            iv = idx_v[ch, pl.ds(off, LANES)]
            for r in range(R):
                rv = jnp.full((LANES,), r, jnp.int32)
                outs[b][r, pl.ds(off, LANES)] = plsc.load_gather(
                    ins[b], [rv, iv])

    for b in range(NBUF):
        start_in(b, b)

    def loop_body(j, _):
        for b in range(NBUF):
            g = j * NBUF + b
            wait_in(b)
            pl.when(j >= 1)(lambda b=b: wait_out(b))
            gather(g, b)
            start_out(g, b)
            pl.when(j < NCHUNK // NBUF - 1)(
                lambda g=g, b=b: start_in(g + NBUF, b))
        return 0

    lax.fori_loop(0, NCHUNK // NBUF, loop_body, 0)
    for b in range(NBUF):
        wait_out(b)


@jax.jit
def _run(x2d, sidx, rand1):
    mesh = plsc.VectorSubcoreMesh(core_axis_name="c", subcore_axis_name="s")
    return pl.kernel(
        _body,
        out_type=jax.ShapeDtypeStruct((ROWS, P), jnp.float32),
        mesh=mesh,
        compiler_params=pltpu.CompilerParams(needs_layout_passes=False,
                                             use_tc_tiling_on_sc=True),
        scratch_types=[
            pltpu.VMEM((LANES,), jnp.int32),
            pltpu.VMEM((C, P), jnp.int32),
            pltpu.VMEM((R, P), jnp.float32),
            pltpu.VMEM((R, P), jnp.float32),
            pltpu.VMEM((R, P), jnp.float32),
            pltpu.VMEM((R, P), jnp.float32),
            pltpu.VMEM((R, P), jnp.float32),
            pltpu.VMEM((R, P), jnp.float32),
            pltpu.VMEM((R, P), jnp.float32),
            pltpu.VMEM((R, P), jnp.float32),
            pltpu.SemaphoreType.DMA,
            pltpu.SemaphoreType.DMA,
            pltpu.SemaphoreType.DMA,
            pltpu.SemaphoreType.DMA,
            pltpu.SemaphoreType.DMA,
            pltpu.SemaphoreType.DMA,
            pltpu.SemaphoreType.DMA,
            pltpu.SemaphoreType.DMA,
        ],
    )(x2d, sidx, rand1)


def kernel(X, shuffled_idx, rand_idx):
    rand1 = jnp.full((LANES,), rand_idx, jnp.int32)
    out = _run(X.reshape(ROWS, P), shuffled_idx, rand1)
    return out.reshape(B, C, E, P)
